# baseline (speedup 1.0000x reference)
import functools

import jax
import jax.numpy as jnp
from jax import lax
from jax.experimental import pallas as pl
from jax.experimental.pallas import tpu as pltpu
from jax.experimental.pallas import tpu_sc as plsc

_MAXLEN = 2048
_EMB = 64
_PADW = 128
_NC, _NS = 2, 16
_NW = _NC * _NS
_CHUNK = 128
_SUB = 128
_NSUB = _CHUNK // _SUB
_NBUF = 2
_L = 16


@functools.cache
def _make_sc_gather(S: int):
    rows_per_w = S // _NW
    jblocks = S // _CHUNK
    nchunk = rows_per_w * jblocks
    assert nchunk % _NBUF == 0
    mesh = plsc.VectorSubcoreMesh(
        core_axis_name="c", subcore_axis_name="s",
        num_cores=_NC, num_subcores=_NS,
    )

    @functools.partial(
        pl.kernel,
        out_type=jax.ShapeDtypeStruct((S, _EMB, S), jnp.float32),
        mesh=mesh,
        compiler_params=pltpu.CompilerParams(needs_layout_passes=False),
        scratch_types=[
            pltpu.VMEM((_NBUF, _CHUNK), jnp.int32),
            pltpu.VMEM((_NBUF, _NSUB, _SUB), jnp.int32),
            pltpu.VMEM((_NBUF, _CHUNK, _PADW), jnp.float32),
            pltpu.VMEM((_NBUF, _EMB, _CHUNK), jnp.float32),
            pltpu.VMEM_SHARED((2 * _MAXLEN, _PADW), jnp.float32),
            pltpu.SemaphoreType.DMA((_NBUF,)),
            pltpu.SemaphoreType.DMA((_NBUF,)),
            pltpu.SemaphoreType.DMA((_NBUF,)),
        ],
    )
    def gather_kernel(pos_hbm, tab_hbm, out_hbm, idx_raw, idx2, rows,
                      plane, tab_sh, isem, gsem, osem):
        wid = lax.axis_index("s") * _NC + lax.axis_index("c")
        i0 = wid * rows_per_w

        @pl.when(lax.axis_index("s") == 0)
        def _():
            pltpu.sync_copy(tab_hbm, tab_sh)
        plsc.subcore_barrier()

        def coords(g):
            return i0 + g // jblocks, (g % jblocks) * _CHUNK

        def idx_copy(g, b):
            i, j0 = coords(g)
            return pltpu.make_async_copy(
                pos_hbm.at[pl.ds(i * S + j0, _CHUNK)],
                idx_raw.at[b], isem.at[b])

        def gat_copy(b, j):
            return pltpu.make_async_copy(
                tab_sh.at[idx2.at[b, j]],
                rows.at[b, pl.ds(j * _SUB, _SUB)], gsem.at[b])

        def out_copy(g, b):
            i, j0 = coords(g)
            return pltpu.make_async_copy(
                plane.at[b],
                out_hbm.at[i, pl.ds(0, _EMB), pl.ds(j0, _CHUNK)], osem.at[b])

        for b in range(_NBUF):
            idx_copy(b, b).start()

        def pair_body(p, carry):
            for b in range(_NBUF):
                g = p * _NBUF + b
                idx_copy(g, b).wait()
                for i in range(_CHUNK // _L):
                    v = idx_raw[b, pl.ds(i * _L, _L)]
                    v = jnp.minimum(v, _MAXLEN - 1)
                    v = jnp.maximum(v, -_MAXLEN)
                    v = v + _MAXLEN
                    r, c = divmod(i * _L, _SUB)
                    idx2[b, r, pl.ds(c, _L)] = v

                for j in range(_NSUB):
                    gat_copy(b, j).start()

                @pl.when(g + _NBUF < nchunk)
                def _():
                    idx_copy(g + _NBUF, b).start()

                for j in range(_NSUB):
                    gat_copy(b, j).wait()

                @pl.when(p > 0)
                def _():
                    out_copy(g - _NBUF, b).wait()

                def jb_body(jb, carry2):
                    idx_j = jb * _L + lax.iota(jnp.int32, _L)
                    rows_b = rows.at[b]
                    for e in range(_EMB):
                        col = plsc.load_gather(
                            rows_b, [idx_j, jnp.full((_L,), e, jnp.int32)])
                        plane[b, e, pl.ds(jb * _L, _L)] = col
                    return carry2

                lax.fori_loop(0, _CHUNK // _L, jb_body, 0)
                out_copy(g, b).start()
            return carry

        lax.fori_loop(0, nchunk // _NBUF, pair_body, 0)
        for b in range(_NBUF):
            out_copy(nchunk - _NBUF + b, b).wait()

    return gather_kernel


@jax.jit
def kernel(positions, pe_k_weight):
    seq_a, seq_b = positions.shape
    pos_flat = positions.reshape(seq_a * seq_b).astype(jnp.int32)
    tab_pad = jnp.pad(pe_k_weight, ((0, 0), (0, _PADW - _EMB)))
    out = _make_sc_gather(seq_a)(pos_flat, tab_pad)
    return jnp.swapaxes(out, 1, 2)


# speedup vs baseline: 1.3347x; 1.3347x over previous
import functools

import jax
import jax.numpy as jnp
from jax import lax
from jax.experimental import pallas as pl
from jax.experimental.pallas import tpu as pltpu
from jax.experimental.pallas import tpu_sc as plsc

_MAXLEN = 2048
_EMB = 64
_PADW = 128
_NC, _NS = 2, 16
_NW = _NC * _NS
_CHUNK = 128
_SUB = 128
_NSUB = _CHUNK // _SUB
_NBUF = 2
_L = 16


@functools.cache
def _make_sc_gather(S: int):
    rows_per_w = S // _NW
    jblocks = S // _CHUNK
    nchunk = rows_per_w * jblocks
    assert nchunk % _NBUF == 0
    mesh = plsc.VectorSubcoreMesh(
        core_axis_name="c", subcore_axis_name="s",
        num_cores=_NC, num_subcores=_NS,
    )

    @functools.partial(
        pl.kernel,
        out_type=jax.ShapeDtypeStruct((S, _EMB, S), jnp.float32),
        mesh=mesh,
        compiler_params=pltpu.CompilerParams(needs_layout_passes=False),
        scratch_types=[
            pltpu.VMEM((_NBUF, _CHUNK), jnp.int32),
            pltpu.VMEM((_NBUF, _NSUB, _SUB), jnp.int32),
            pltpu.VMEM((_NBUF, _CHUNK, _PADW), jnp.float32),
            pltpu.VMEM((_NBUF, _EMB, _CHUNK), jnp.float32),
            pltpu.VMEM_SHARED((2 * _MAXLEN, _PADW), jnp.float32),
            pltpu.SemaphoreType.DMA((_NBUF,)),
            pltpu.SemaphoreType.DMA((_NBUF,)),
            pltpu.SemaphoreType.DMA((_NBUF,)),
        ],
    )
    def gather_kernel(pos_hbm, tab_hbm, out_hbm, idx_raw, idx2, rows,
                      plane, tab_sh, isem, gsem, osem):
        wid = lax.axis_index("s") * _NC + lax.axis_index("c")
        i0 = wid * rows_per_w

        @pl.when(lax.axis_index("s") == 0)
        def _():
            pltpu.sync_copy(tab_hbm, tab_sh)
        plsc.subcore_barrier()

        def coords(g):
            return i0 + g // jblocks, (g % jblocks) * _CHUNK

        def idx_copy(g, b):
            i, j0 = coords(g)
            return pltpu.make_async_copy(
                pos_hbm.at[pl.ds(i * S + j0, _CHUNK)],
                idx_raw.at[b], isem.at[b])

        def gat_copy(b, j):
            return pltpu.make_async_copy(
                tab_sh.at[idx2.at[b, j]],
                rows.at[b, pl.ds(j * _SUB, _SUB)], gsem.at[b])

        def out_copy(g, b):
            i, j0 = coords(g)
            return pltpu.make_async_copy(
                plane.at[b],
                out_hbm.at[i, pl.ds(0, _EMB), pl.ds(j0, _CHUNK)], osem.at[b])

        for b in range(_NBUF):
            idx_copy(b, b).start()

        def pair_body(p, carry):
            for b in range(_NBUF):
                g = p * _NBUF + b
                idx_copy(g, b).wait()
                for i in range(_CHUNK // _L):
                    v = idx_raw[b, pl.ds(i * _L, _L)]
                    v = jnp.minimum(v, _MAXLEN - 1)
                    v = jnp.maximum(v, -_MAXLEN)
                    v = v + _MAXLEN
                    r, c = divmod(i * _L, _SUB)
                    idx2[b, r, pl.ds(c, _L)] = v

                for j in range(_NSUB):
                    gat_copy(b, j).start()

                @pl.when(g + _NBUF < nchunk)
                def _():
                    idx_copy(g + _NBUF, b).start()

                for j in range(_NSUB):
                    gat_copy(b, j).wait()

                @pl.when(p > 0)
                def _():
                    out_copy(g - _NBUF, b).wait()

                @plsc.parallel_loop(0, _CHUNK // _L, unroll=2)
                def _(jb):
                    idx_j = jb * _L + lax.iota(jnp.int32, _L)
                    rows_b = rows.at[b]
                    for e in range(_EMB):
                        col = plsc.load_gather(
                            rows_b, [idx_j, jnp.full((_L,), e, jnp.int32)])
                        plane[b, e, pl.ds(jb * _L, _L)] = col
                out_copy(g, b).start()
            return carry

        lax.fori_loop(0, nchunk // _NBUF, pair_body, 0)
        for b in range(_NBUF):
            out_copy(nchunk - _NBUF + b, b).wait()

    return gather_kernel


@jax.jit
def kernel(positions, pe_k_weight):
    seq_a, seq_b = positions.shape
    pos_flat = positions.reshape(seq_a * seq_b).astype(jnp.int32)
    tab_pad = jnp.pad(pe_k_weight, ((0, 0), (0, _PADW - _EMB)))
    out = _make_sc_gather(seq_a)(pos_flat, tab_pad)
    return jnp.swapaxes(out, 1, 2)


# speedup vs baseline: 3.8437x; 2.8799x over previous
import functools

import jax
import jax.numpy as jnp
from jax import lax
from jax.experimental import pallas as pl
from jax.experimental.pallas import tpu as pltpu
from jax.experimental.pallas import tpu_sc as plsc

_MAXLEN = 2048
_EMB = 64
_NC, _NS = 2, 16
_NW = _NC * _NS
_CHUNK = 128
_SUB = 128
_NSUB = _CHUNK // _SUB
_NBUF = 2
_PADW = 128


@functools.cache
def _make_sc_gather(B: int):
    bpw = B // _NW
    nchunk = bpw // _CHUNK
    assert nchunk % _NBUF == 0
    mesh = plsc.VectorSubcoreMesh(
        core_axis_name="c", subcore_axis_name="s",
        num_cores=_NC, num_subcores=_NS,
    )

    @functools.partial(
        pl.kernel,
        out_type=jax.ShapeDtypeStruct((B, _EMB), jnp.float32),
        mesh=mesh,
        scratch_types=[
            pltpu.VMEM((_NBUF, _CHUNK), jnp.int32),
            pltpu.VMEM((_NBUF, _NSUB, _SUB), jnp.int32),
            pltpu.VMEM((_NBUF, _CHUNK, _PADW), jnp.float32),
            pltpu.VMEM((_NBUF, _CHUNK, _EMB), jnp.float32),
            pltpu.VMEM_SHARED((2 * _MAXLEN, _PADW), jnp.float32),
            pltpu.SemaphoreType.DMA((_NBUF,)),
            pltpu.SemaphoreType.DMA((_NBUF,)),
            pltpu.SemaphoreType.DMA((_NBUF,)),
        ],
    )
    def gather_kernel(pos_hbm, tab_hbm, out_hbm, idx_raw, idx2, rows,
                      rows64, tab_sh, isem, gsem, osem):
        wid = lax.axis_index("s") * _NC + lax.axis_index("c")
        base = wid * bpw

        @pl.when(lax.axis_index("s") == 0)
        def _():
            pltpu.sync_copy(tab_hbm, tab_sh)
        plsc.subcore_barrier()

        def idx_copy(g, b):
            return pltpu.make_async_copy(
                pos_hbm.at[pl.ds(base + g * _CHUNK, _CHUNK)],
                idx_raw.at[b], isem.at[b])

        def gat_copy(b, j):
            return pltpu.make_async_copy(
                tab_sh.at[idx2.at[b, j]],
                rows.at[b, pl.ds(j * _SUB, _SUB)], gsem.at[b])

        def out_copy(g, b):
            return pltpu.make_async_copy(
                rows64.at[b],
                out_hbm.at[pl.ds(base + g * _CHUNK, _CHUNK)], osem.at[b])

        for b in range(_NBUF):
            idx_copy(b, b).start()

        def pair_body(p, carry):
            for b in range(_NBUF):
                g = p * _NBUF + b
                idx_copy(g, b).wait()
                for i in range(_CHUNK // 16):
                    v = idx_raw[b, pl.ds(i * 16, 16)]
                    v = jnp.minimum(v, _MAXLEN - 1)
                    v = jnp.maximum(v, -_MAXLEN)
                    v = v + _MAXLEN
                    r, c = divmod(i * 16, _SUB)
                    idx2[b, r, pl.ds(c, 16)] = v

                @pl.when(p > 0)
                def _():
                    out_copy(g - _NBUF, b).wait()

                for j in range(_NSUB):
                    gat_copy(b, j).start()

                @pl.when(g + _NBUF < nchunk)
                def _():
                    idx_copy(g + _NBUF, b).start()

                for j in range(_NSUB):
                    gat_copy(b, j).wait()

                @plsc.parallel_loop(0, _CHUNK, unroll=2)
                def _(j):
                    for e0 in range(_EMB // 16):
                        rows64[b, j, pl.ds(e0 * 16, 16)] = (
                            rows[b, j, pl.ds(e0 * 16, 16)])
                out_copy(g, b).start()
            return carry

        lax.fori_loop(0, nchunk // _NBUF, pair_body, 0)
        for b in range(_NBUF):
            out_copy(nchunk - _NBUF + b, b).wait()

    return gather_kernel


@jax.jit
def kernel(positions, pe_k_weight):
    seq_a, seq_b = positions.shape
    B = seq_a * seq_b
    pos_flat = positions.reshape(B).astype(jnp.int32)
    tab_pad = jnp.pad(pe_k_weight, ((0, 0), (0, _PADW - _EMB)))
    out = _make_sc_gather(B)(pos_flat, tab_pad)
    return out.reshape(seq_a, seq_b, _EMB)


# speedup vs baseline: 3.8566x; 1.0033x over previous
import functools

import jax
import jax.numpy as jnp
from jax import lax
from jax.experimental import pallas as pl
from jax.experimental.pallas import tpu as pltpu
from jax.experimental.pallas import tpu_sc as plsc

_MAXLEN = 2048
_EMB = 64
_NC, _NS = 2, 16
_NW = _NC * _NS
_CHUNK = 128
_SUB = 128
_NSUB = _CHUNK // _SUB
_NBUF = 2
_PADW = 128


@functools.cache
def _make_sc_gather(B: int):
    bpw = B // _NW
    nchunk = bpw // _CHUNK
    assert nchunk % _NBUF == 0
    mesh = plsc.VectorSubcoreMesh(
        core_axis_name="c", subcore_axis_name="s",
        num_cores=_NC, num_subcores=_NS,
    )

    @functools.partial(
        pl.kernel,
        out_type=jax.ShapeDtypeStruct((B, _EMB), jnp.float32),
        mesh=mesh,
        scratch_types=[
            pltpu.VMEM((_NBUF, _CHUNK), jnp.int32),
            pltpu.VMEM((_NBUF, _NSUB, _SUB), jnp.int32),
            pltpu.VMEM((_NBUF, _CHUNK, _PADW), jnp.float32),
            pltpu.VMEM((_NBUF, _CHUNK, _EMB), jnp.float32),
            pltpu.VMEM_SHARED((2 * _MAXLEN, _PADW), jnp.float32),
            pltpu.SemaphoreType.DMA((_NBUF,)),
            pltpu.SemaphoreType.DMA((_NBUF,)),
            pltpu.SemaphoreType.DMA((_NBUF,)),
        ],
    )
    def gather_kernel(pos_hbm, tab_hbm, out_hbm, idx_raw, idx2, rows,
                      rows64, tab_sh, isem, gsem, osem):
        wid = lax.axis_index("s") * _NC + lax.axis_index("c")
        base = wid * bpw

        @pl.when(lax.axis_index("s") == 0)
        def _():
            pltpu.sync_copy(tab_hbm, tab_sh)
        plsc.subcore_barrier()

        def idx_copy(g, b):
            return pltpu.make_async_copy(
                pos_hbm.at[pl.ds(base + g * _CHUNK, _CHUNK)],
                idx_raw.at[b], isem.at[b])

        def gat_copy(b, j):
            return pltpu.make_async_copy(
                tab_sh.at[idx2.at[b, j]],
                rows.at[b, pl.ds(j * _SUB, _SUB)], gsem.at[b])

        def out_copy(g, b):
            return pltpu.make_async_copy(
                rows64.at[b],
                out_hbm.at[pl.ds(base + g * _CHUNK, _CHUNK)], osem.at[b])

        for b in range(_NBUF):
            idx_copy(b, b).start()

        def pair_body(p, carry):
            for b in range(_NBUF):
                g = p * _NBUF + b
                idx_copy(g, b).wait()
                for i in range(_CHUNK // 16):
                    v = idx_raw[b, pl.ds(i * 16, 16)]
                    v = jnp.minimum(v, _MAXLEN - 1)
                    v = jnp.maximum(v, -_MAXLEN)
                    v = v + _MAXLEN
                    r, c = divmod(i * 16, _SUB)
                    idx2[b, r, pl.ds(c, 16)] = v

                @pl.when(p > 0)
                def _():
                    out_copy(g - _NBUF, b).wait()

                for j in range(_NSUB):
                    gat_copy(b, j).start()

                @pl.when(g + _NBUF < nchunk)
                def _():
                    idx_copy(g + _NBUF, b).start()

                for j in range(_NSUB):
                    gat_copy(b, j).wait()

                @plsc.parallel_loop(0, _CHUNK, unroll=4)
                def _(j):
                    for e0 in range(_EMB // 16):
                        rows64[b, j, pl.ds(e0 * 16, 16)] = (
                            rows[b, j, pl.ds(e0 * 16, 16)])
                out_copy(g, b).start()
            return carry

        lax.fori_loop(0, nchunk // _NBUF, pair_body, 0)
        for b in range(_NBUF):
            out_copy(nchunk - _NBUF + b, b).wait()

    return gather_kernel


@jax.jit
def kernel(positions, pe_k_weight):
    seq_a, seq_b = positions.shape
    B = seq_a * seq_b
    pos_flat = positions.reshape(B).astype(jnp.int32)
    tab_pad = jnp.pad(pe_k_weight, ((0, 0), (0, _PADW - _EMB)))
    out = _make_sc_gather(B)(pos_flat, tab_pad)
    return out.reshape(seq_a, seq_b, _EMB)
